# baseline (device time: 70397 ns/iter reference)
import jax
import jax.numpy as jnp
from jax import lax
from jax.experimental import pallas as pl
from jax.experimental.pallas import tpu as pltpu

T = 1024
D = 2048
V_LOCAL = 16384
V_TILE = 1024
N_TILES = (V_LOCAL // 2) // V_TILE


def kernel(x, W, labels):
    labels2d = labels.reshape(T, 1)

    def body(x_ref, w_hbm, lab_ref, out_ref,
             xb_ref, wbuf, send_ref, recv_ref, wsem, ssem, rsem):
        my_x = lax.axis_index("x")
        my_y = lax.axis_index("y")

        col0 = my_x * (N_TILES * V_TILE)

        def tile_copy(slot, t):
            return pltpu.make_async_copy(
                w_hbm.at[:, pl.ds(col0 + t * V_TILE, V_TILE)],
                wbuf.at[slot],
                wsem.at[slot],
            )

        tile_copy(0, 0).start()
        xb_ref[...] = x_ref[...].astype(jnp.bfloat16)

        peers = [(1 - my_x, my_y), (my_x, 1 - my_y), (1 - my_x, 1 - my_y)]
        barrier_sem = pltpu.get_barrier_semaphore()
        for p in peers:
            pl.semaphore_signal(
                barrier_sem, inc=1, device_id=p,
                device_id_type=pl.DeviceIdType.MESH,
            )
        pl.semaphore_wait(barrier_sem, 3)

        s_acc = jnp.zeros((T, 1), jnp.float32)
        t_acc = jnp.zeros((T, 1), jnp.float32)
        cols = lax.broadcasted_iota(jnp.int32, (T, V_TILE), 1)

        def process(logits, t):
            nonlocal s_acc, t_acc
            s_acc = s_acc + jnp.sum(jnp.exp(logits), axis=1, keepdims=True)
            base = my_y * V_LOCAL + col0 + t * V_TILE
            loc = lab_ref[...] - base
            t_acc = t_acc + jnp.sum(
                jnp.where(cols == loc, logits, 0.0), axis=1, keepdims=True
            )

        logits_prev = None
        for t in range(N_TILES):
            slot = t % 2
            if t + 1 < N_TILES:
                tile_copy(1 - slot, t + 1).start()
            tile_copy(slot, t).wait()
            logits = jnp.dot(
                xb_ref[...],
                wbuf[slot].astype(jnp.bfloat16),
                preferred_element_type=jnp.float32,
            )
            if logits_prev is not None:
                process(logits_prev, t - 1)
            logits_prev = logits
        process(logits_prev, N_TILES - 1)

        send_ref[:, 0:1] = s_acc
        send_ref[:, 1:2] = t_acc
        rdmas = [
            pltpu.make_async_remote_copy(
                src_ref=send_ref,
                dst_ref=recv_ref.at[i],
                send_sem=ssem.at[i],
                recv_sem=rsem.at[i],
                device_id=p,
                device_id_type=pl.DeviceIdType.MESH,
            )
            for i, p in enumerate(peers)
        ]
        for r in rdmas:
            r.start()
        for r in rdmas:
            r.wait()
        s_tot = s_acc + recv_ref[0, :, 0:1] + recv_ref[1, :, 0:1] + recv_ref[2, :, 0:1]
        t_tot = t_acc + recv_ref[0, :, 1:2] + recv_ref[1, :, 1:2] + recv_ref[2, :, 1:2]
        out_ref[...] = jnp.log(s_tot) - t_tot

    out = pl.pallas_call(
        body,
        in_specs=[
            pl.BlockSpec(memory_space=pltpu.MemorySpace.VMEM),
            pl.BlockSpec(memory_space=pltpu.MemorySpace.HBM),
            pl.BlockSpec(memory_space=pltpu.MemorySpace.VMEM),
        ],
        out_specs=pl.BlockSpec(memory_space=pltpu.MemorySpace.VMEM),
        out_shape=jax.ShapeDtypeStruct((T, 1), jnp.float32),
        scratch_shapes=[
            pltpu.VMEM((T, D), jnp.bfloat16),
            pltpu.VMEM((2, D, V_TILE), jnp.float32),
            pltpu.VMEM((T, 2), jnp.float32),
            pltpu.VMEM((3, T, 2), jnp.float32),
            pltpu.SemaphoreType.DMA((2,)),
            pltpu.SemaphoreType.DMA((3,)),
            pltpu.SemaphoreType.DMA((3,)),
        ],
        compiler_params=pltpu.CompilerParams(
            vmem_limit_bytes=96 * 1024 * 1024,
            collective_id=0,
        ),
    )(x, W, labels2d)
    return out.reshape(T)
